# baseline (speedup 1.0000x reference)
import functools

import jax
import jax.numpy as jnp
from jax import lax
from jax.experimental import pallas as pl
from jax.experimental.pallas import tpu as pltpu
from jax.experimental.pallas import tpu_sc as plsc

_LANES = 128
_K = 5


@functools.lru_cache(maxsize=None)
def _build(n_idx, vocab, dim):
    info = plsc.get_sparse_core_info()
    nc, ns = info.num_cores, info.num_subcores
    nw = nc * ns
    rows_total = n_idx // _LANES
    rows_per_w = rows_total // nw
    nchunk = rows_per_w // _K
    npair = nchunk // 2

    mesh = plsc.VectorSubcoreMesh(core_axis_name="c", subcore_axis_name="s")

    @functools.partial(
        pl.kernel,
        mesh=mesh,
        out_type=jax.ShapeDtypeStruct((rows_total, _LANES, dim), jnp.float32),
        scratch_types=[
            pltpu.VMEM((rows_per_w, _LANES), jnp.int32),
            pltpu.VMEM((_K, _LANES, dim), jnp.float32),
            pltpu.VMEM((_K, _LANES, dim), jnp.float32),
            pltpu.SemaphoreType.DMA,
            pltpu.SemaphoreType.DMA,
        ],
        compiler_params=pltpu.CompilerParams(use_tc_tiling_on_sc=False),
    )
    def emb(idx_hbm, table_hbm, out_hbm, idx_v, rows0, rows1, sem0, sem1):
        wid = lax.axis_index("s") * nc + lax.axis_index("c")
        row0 = wid * rows_per_w
        pltpu.sync_copy(idx_hbm.at[pl.ds(row0, rows_per_w)], idx_v)

        def fire(chunk, buf, sem):
            for j in range(_K):
                pltpu.async_copy(
                    table_hbm.at[idx_v.at[chunk * _K + j]], buf.at[j], sem)

        def drain(chunk, buf, sem):
            for j in range(_K):
                pltpu.make_async_copy(
                    table_hbm.at[idx_v.at[chunk * _K + j]], buf.at[j],
                    sem).wait()

        def put(chunk, buf):
            pltpu.sync_copy(buf, out_hbm.at[pl.ds(row0 + chunk * _K, _K)])

        fire(0, rows0, sem0)

        def pair(p, _):
            c0 = p * 2
            drain(c0, rows0, sem0)
            fire(c0 + 1, rows1, sem1)
            put(c0, rows0)
            drain(c0 + 1, rows1, sem1)

            @pl.when(p + 1 < npair)
            def _():
                fire(c0 + 2, rows0, sem0)

            put(c0 + 1, rows1)
            return 0

        lax.fori_loop(0, npair, pair, 0)

    return emb


def kernel(inputs, table):
    seq, batch = inputs.shape
    vocab, dim = table.shape
    n_idx = seq * batch
    flat_idx = inputs.reshape(n_idx // _LANES, _LANES)
    out = _build(n_idx, vocab, dim)(flat_idx, table)
    return out.reshape(seq, batch, dim)
